# baseline (device time: 1382876 ns/iter reference)
import jax
import jax.numpy as jnp
from jax import lax
from jax.experimental import pallas as pl
from jax.experimental.pallas import tpu as pltpu

N_DEV = 32
B, SQ, SKV = 2, 512, 512
H_LOC, DH = 8, 64
DM = 768
BLK = 64


def kernel(x, Wq, K_ext, V_ext, Wo):
    my = lax.axis_index("i")
    K_loc = lax.dynamic_slice_in_dim(K_ext, my * H_LOC, H_LOC, axis=2)
    V_loc = lax.dynamic_slice_in_dim(V_ext, my * H_LOC, H_LOC, axis=2)
    K_loc = jnp.transpose(K_loc, (0, 2, 1, 3))
    V_loc = jnp.transpose(V_loc, (0, 2, 1, 3))

    def body(x_ref, wq_ref, k_ref, v_ref, wo_ref, out_ref,
             comm_ref, send_sems, recv_sems, credit_sem):
        my_pos = lax.axis_index("i")
        left = (my_pos + N_DEV - 1) % N_DEV
        right = (my_pos + 1) % N_DEV

        barrier_sem = pltpu.get_barrier_semaphore()
        for nbr in (left, right):
            pl.semaphore_signal(
                barrier_sem, inc=1,
                device_id=(nbr,), device_id_type=pl.DeviceIdType.MESH,
            )
        pl.semaphore_wait(barrier_sem, 2)

        rowblk = lax.broadcasted_iota(jnp.int32, (SQ, SKV), 0) // BLK
        colblk = lax.broadcasted_iota(jnp.int32, (SQ, SKV), 1) // BLK
        mask = colblk <= rowblk

        for b in range(B):
            q_b = jnp.dot(x_ref[b], wq_ref[...],
                          preferred_element_type=jnp.float32)
            ctxs = []
            for h in range(H_LOC):
                q = q_b[:, h * DH:(h + 1) * DH]
                k = k_ref[b, h]
                v = v_ref[b, h]
                s = lax.dot_general(
                    q, k, (((1,), (1,)), ((), ())),
                    preferred_element_type=jnp.float32) * 0.125
                s = jnp.where(mask, s, -1e9)
                m = jnp.max(s, axis=1, keepdims=True)
                p = jnp.exp(s - m)
                p = p / jnp.sum(p, axis=1, keepdims=True)
                ctxs.append(jnp.dot(p, v, preferred_element_type=jnp.float32))
            ctx = jnp.concatenate(ctxs, axis=1)
            part = jnp.dot(ctx, wo_ref[...],
                           preferred_element_type=jnp.float32)
            out_ref[b] = part
            comm_ref[0, b] = part

        for hop in range(N_DEV - 1):
            s_slot = hop % 2
            r_slot = (hop + 1) % 2
            if hop >= 1:
                pl.semaphore_wait(credit_sem, 1)
            rdma = pltpu.make_async_remote_copy(
                src_ref=comm_ref.at[s_slot],
                dst_ref=comm_ref.at[r_slot],
                send_sem=send_sems.at[s_slot],
                recv_sem=recv_sems.at[r_slot],
                device_id=(right,),
                device_id_type=pl.DeviceIdType.MESH,
            )
            rdma.start()
            rdma.wait()
            for b in range(B):
                out_ref[b] += comm_ref[r_slot, b]
            if hop < N_DEV - 2:
                pl.semaphore_signal(
                    credit_sem, inc=1,
                    device_id=(left,), device_id_type=pl.DeviceIdType.MESH,
                )

    return pl.pallas_call(
        body,
        out_shape=jax.ShapeDtypeStruct((B, SQ, DM), jnp.float32),
        in_specs=[pl.BlockSpec(memory_space=pltpu.VMEM)] * 5,
        out_specs=pl.BlockSpec(memory_space=pltpu.VMEM),
        scratch_shapes=[
            pltpu.VMEM((2, B, SQ, DM), jnp.float32),
            pltpu.SemaphoreType.DMA((2,)),
            pltpu.SemaphoreType.DMA((2,)),
            pltpu.SemaphoreType.REGULAR,
        ],
        compiler_params=pltpu.CompilerParams(collective_id=0),
    )(x, Wq, K_loc, V_loc, Wo)


# device time: 205852 ns/iter; 6.7178x vs baseline; 6.7178x over previous
import jax
import jax.numpy as jnp
from jax import lax
from jax.experimental import pallas as pl
from jax.experimental.pallas import tpu as pltpu

N_DEV = 32
B, SQ, SKV = 2, 512, 512
H_LOC, DH = 8, 64
DM = 768
BLK = 64
ROWS = B * SQ

RS_HALF = [512, 256, 128, 64, 32]
RS_SOFF = [0, 512, 768, 896, 960]
AG_SZ = [32, 64, 128, 256, 512]


def kernel(x, Wq, K_ext, V_ext, Wo):
    my = lax.axis_index("i")
    K_loc = lax.dynamic_slice_in_dim(K_ext, my * H_LOC, H_LOC, axis=2)
    V_loc = lax.dynamic_slice_in_dim(V_ext, my * H_LOC, H_LOC, axis=2)
    K_loc = jnp.transpose(K_loc, (0, 2, 1, 3))
    V_loc = jnp.transpose(V_loc, (0, 2, 1, 3))
    x2 = x.reshape(ROWS, DM)

    def body(x_ref, wq_ref, k_ref, v_ref, wo_ref, out_ref,
             stage_ref, send_sems, recv_sems):
        p = lax.axis_index("i")
        z = p // 8
        q = p % 8
        y = q // 2
        xb = (q % 2) ^ (y % 2)

        def pos_of(xb_, y_, z_):
            return 8 * z_ + 2 * y_ + (xb_ ^ (y_ % 2))

        levels = [
            (xb, pos_of(xb ^ 1, y, z)),
            (y % 2, pos_of(xb, y ^ 1, z)),
            (z % 2, pos_of(xb, y, z ^ 1)),
            (y // 2, pos_of(xb, y ^ 2, z)),
            (z // 2, pos_of(xb, y, z ^ 2)),
        ]

        barrier_sem = pltpu.get_barrier_semaphore()
        for _, partner in levels:
            pl.semaphore_signal(
                barrier_sem, inc=1,
                device_id=(partner,), device_id_type=pl.DeviceIdType.MESH,
            )
        pl.semaphore_wait(barrier_sem, len(levels))

        rowblk = lax.broadcasted_iota(jnp.int32, (SQ, SKV), 0) // BLK
        colblk = lax.broadcasted_iota(jnp.int32, (SQ, SKV), 1) // BLK
        mask = colblk <= rowblk

        for b in range(B):
            q_b = jnp.dot(x_ref[pl.ds(b * SQ, SQ), :], wq_ref[...],
                          preferred_element_type=jnp.float32)
            ctxs = []
            for h in range(H_LOC):
                qh = q_b[:, h * DH:(h + 1) * DH]
                kh = k_ref[b, h]
                vh = v_ref[b, h]
                s = lax.dot_general(
                    qh, kh, (((1,), (1,)), ((), ())),
                    preferred_element_type=jnp.float32) * 0.125
                s = jnp.where(mask, s, -1e9)
                m = jnp.max(s, axis=1, keepdims=True)
                pr = jnp.exp(s - m)
                pr = pr / jnp.sum(pr, axis=1, keepdims=True)
                ctxs.append(jnp.dot(pr, vh, preferred_element_type=jnp.float32))
            ctx = jnp.concatenate(ctxs, axis=1)
            out_ref[pl.ds(b * SQ, SQ), :] = jnp.dot(
                ctx, wo_ref[...], preferred_element_type=jnp.float32)

        off = jnp.int32(0)
        for k, (bit, partner) in enumerate(levels):
            half = RS_HALF[k]
            give = off + (1 - bit) * half
            keep = off + bit * half
            rdma = pltpu.make_async_remote_copy(
                src_ref=out_ref.at[pl.ds(give, half), :],
                dst_ref=stage_ref.at[pl.ds(RS_SOFF[k], half), :],
                send_sem=send_sems.at[k],
                recv_sem=recv_sems.at[k],
                device_id=(partner,),
                device_id_type=pl.DeviceIdType.MESH,
            )
            rdma.start()
            rdma.wait()
            out_ref[pl.ds(keep, half), :] += stage_ref[pl.ds(RS_SOFF[k], half), :]
            off = keep

        for j in range(len(levels)):
            k = len(levels) - 1 - j
            bit, partner = levels[k]
            sz = AG_SZ[j]
            rdma = pltpu.make_async_remote_copy(
                src_ref=out_ref.at[pl.ds(off, sz), :],
                dst_ref=out_ref.at[pl.ds(off, sz), :],
                send_sem=send_sems.at[5 + j],
                recv_sem=recv_sems.at[5 + j],
                device_id=(partner,),
                device_id_type=pl.DeviceIdType.MESH,
            )
            rdma.start()
            rdma.wait()
            off = off - bit * sz

    out2 = pl.pallas_call(
        body,
        out_shape=jax.ShapeDtypeStruct((ROWS, DM), jnp.float32),
        in_specs=[pl.BlockSpec(memory_space=pltpu.VMEM)] * 5,
        out_specs=pl.BlockSpec(memory_space=pltpu.VMEM),
        scratch_shapes=[
            pltpu.VMEM((992, DM), jnp.float32),
            pltpu.SemaphoreType.DMA((10,)),
            pltpu.SemaphoreType.DMA((10,)),
        ],
        compiler_params=pltpu.CompilerParams(collective_id=0),
    )(x2, Wq, K_loc, V_loc, Wo)
    return out2.reshape(B, SQ, DM)
